# baseline (speedup 1.0000x reference)
import functools

import jax
import jax.numpy as jnp
from jax import lax
from jax.experimental import pallas as pl
from jax.experimental.pallas import tpu as pltpu
from jax.experimental.pallas import tpu_sc as plsc

_B = 8
_C_OUT = 64
_STRIDE = 12

_C_TC = 36
_C_SC = _C_OUT - _C_TC


_RING = 4
_LOOK = 2


def _tc_kernel(x_ref, o_ref, buf, rsems, wsems):
    reads = [
        pltpu.make_async_copy(
            x_ref.at[b, pl.ds(0, _C_TC), 0], buf.at[b % _RING], rsems.at[b % _RING]
        )
        for b in range(_B)
    ]
    writes = [
        pltpu.make_async_copy(
            buf.at[b % _RING], o_ref.at[b, pl.ds(0, _C_TC)], wsems.at[b % _RING]
        )
        for b in range(_B)
    ]
    for i in range(_B + _LOOK):
        if i < _B:
            if i >= _RING:
                writes[i - _RING].wait()
            reads[i].start()
        j = i - _LOOK
        if 0 <= j < _B:
            reads[j].wait()
            writes[j].start()
    for i in range(_B - _RING, _B):
        writes[i].wait()


def _tc_copy(x):
    xv = x.reshape(_B, _C_OUT, _STRIDE, 224, 224)
    return pl.pallas_call(
        _tc_kernel,
        in_specs=[pl.BlockSpec(memory_space=pl.ANY)],
        out_specs=pl.BlockSpec(memory_space=pl.ANY),
        out_shape=jax.ShapeDtypeStruct((_B, _C_OUT, 224, 224), jnp.float32),
        scratch_shapes=[
            pltpu.VMEM((_RING, _C_TC, 224, 224), jnp.float32),
            pltpu.SemaphoreType.DMA((_RING,)),
            pltpu.SemaphoreType.DMA((_RING,)),
        ],
    )(xv)


_NC = 2
_NS = 16
_NW = _NC * _NS
_N_SC = _B * _C_SC
_PER_W = _N_SC // _NW

_mesh = plsc.VectorSubcoreMesh(core_axis_name="c", subcore_axis_name="s")


@functools.partial(
    pl.kernel,
    out_type=jax.ShapeDtypeStruct((_B, _C_SC, 224, 224), jnp.float32),
    mesh=_mesh,
    scratch_types=[
        pltpu.VMEM((2, 224, 224), jnp.float32),
        pltpu.SemaphoreType.DMA((2,)),
        pltpu.SemaphoreType.DMA((2,)),
    ],
)
def _sc_copy(x_hbm, o_hbm, buf, isems, osems):
    wid = lax.axis_index("s") * _NC + lax.axis_index("c")

    def src(k):
        p = wid * _PER_W + k
        return x_hbm.at[p // _C_SC, (_C_TC + p % _C_SC) * _STRIDE]

    def dst(k):
        p = wid * _PER_W + k
        return o_hbm.at[p // _C_SC, p % _C_SC]

    cur_in = pltpu.async_copy(src(0), buf.at[0], isems.at[0])
    prev_out = None
    for k in range(_PER_W):
        s = k % 2
        cur_in.wait()
        if prev_out is not None:
            prev_out.wait()
        if k + 1 < _PER_W:
            cur_in = pltpu.async_copy(src(k + 1), buf.at[1 - s], isems.at[1 - s])
        prev_out = pltpu.async_copy(buf.at[s], dst(k), osems.at[s])
    prev_out.wait()


def kernel(x):
    tc_full = _tc_copy(x)
    sc_part = _sc_copy(x)
    return lax.dynamic_update_slice(tc_full, sc_part, (0, _C_TC, 0, 0))


# speedup vs baseline: 1.0014x; 1.0014x over previous
import functools

import jax
import jax.numpy as jnp
from jax import lax
from jax.experimental import pallas as pl
from jax.experimental.pallas import tpu as pltpu
from jax.experimental.pallas import tpu_sc as plsc

_B = 8
_C_OUT = 64
_STRIDE = 12

_G = 8
_NG_TC = 5
_C_TC = _G * _NG_TC
_C_SC = _C_OUT - _C_TC


_T = _B * _NG_TC
_RING = 8
_LOOK = 4


def _tc_kernel(x_ref, o_ref, buf, rsems, wsems):
    reads = [
        pltpu.make_async_copy(
            x_ref.at[t // _NG_TC, t % _NG_TC, :, 0],
            buf.at[t % _RING],
            rsems.at[t % _RING],
        )
        for t in range(_T)
    ]
    writes = [
        pltpu.make_async_copy(
            buf.at[t % _RING],
            o_ref.at[t // _NG_TC, pl.ds((t % _NG_TC) * _G, _G)],
            wsems.at[t % _RING],
        )
        for t in range(_T)
    ]
    for i in range(_T + _LOOK):
        if i < _T:
            if i >= _RING:
                writes[i - _RING].wait()
            reads[i].start()
        j = i - _LOOK
        if 0 <= j < _T:
            reads[j].wait()
            writes[j].start()
    for i in range(_T - _RING, _T):
        writes[i].wait()


def _tc_copy(x):
    xv = x.reshape(_B, _G, _G, _STRIDE, 224, 224)
    return pl.pallas_call(
        _tc_kernel,
        in_specs=[pl.BlockSpec(memory_space=pl.ANY)],
        out_specs=pl.BlockSpec(memory_space=pl.ANY),
        out_shape=jax.ShapeDtypeStruct((_B, _C_OUT, 224, 224), jnp.float32),
        scratch_shapes=[
            pltpu.VMEM((_RING, _G, 224, 224), jnp.float32),
            pltpu.SemaphoreType.DMA((_RING,)),
            pltpu.SemaphoreType.DMA((_RING,)),
        ],
    )(xv)


_NC = 2
_NS = 16
_NW = _NC * _NS
_N_SC = _B * _C_SC
_PER_W = _N_SC // _NW

_mesh = plsc.VectorSubcoreMesh(core_axis_name="c", subcore_axis_name="s")


@functools.partial(
    pl.kernel,
    out_type=jax.ShapeDtypeStruct((_B, _C_SC, 224, 224), jnp.float32),
    mesh=_mesh,
    scratch_types=[
        pltpu.VMEM((2, 224, 224), jnp.float32),
        pltpu.SemaphoreType.DMA((2,)),
        pltpu.SemaphoreType.DMA((2,)),
    ],
)
def _sc_copy(x_hbm, o_hbm, buf, isems, osems):
    wid = lax.axis_index("s") * _NC + lax.axis_index("c")

    def src(k):
        p = wid * _PER_W + k
        return x_hbm.at[p // _C_SC, (_C_TC + p % _C_SC) * _STRIDE]

    def dst(k):
        p = wid * _PER_W + k
        return o_hbm.at[p // _C_SC, p % _C_SC]

    cur_in = pltpu.async_copy(src(0), buf.at[0], isems.at[0])
    prev_out = None
    for k in range(_PER_W):
        s = k % 2
        cur_in.wait()
        if prev_out is not None:
            prev_out.wait()
        if k + 1 < _PER_W:
            cur_in = pltpu.async_copy(src(k + 1), buf.at[1 - s], isems.at[1 - s])
        prev_out = pltpu.async_copy(buf.at[s], dst(k), osems.at[s])
    prev_out.wait()


def _merge_kernel(sc_ref, tc_ref, o_ref, buf, rsems, wsems):
    reads = [
        pltpu.make_async_copy(sc_ref.at[b], buf.at[b % 2], rsems.at[b % 2])
        for b in range(_B)
    ]
    writes = [
        pltpu.make_async_copy(
            buf.at[b % 2], o_ref.at[b, pl.ds(_C_TC, _C_SC)], wsems.at[b % 2]
        )
        for b in range(_B)
    ]
    for i in range(_B + 1):
        if i < _B:
            if i >= 2:
                writes[i - 2].wait()
            reads[i].start()
        j = i - 1
        if 0 <= j < _B:
            reads[j].wait()
            writes[j].start()
    for i in range(_B - 2, _B):
        writes[i].wait()


def _merge(tc_full, sc_part):
    return pl.pallas_call(
        _merge_kernel,
        in_specs=[
            pl.BlockSpec(memory_space=pl.ANY),
            pl.BlockSpec(memory_space=pl.ANY),
        ],
        out_specs=pl.BlockSpec(memory_space=pl.ANY),
        out_shape=jax.ShapeDtypeStruct((_B, _C_OUT, 224, 224), jnp.float32),
        input_output_aliases={1: 0},
        scratch_shapes=[
            pltpu.VMEM((2, _C_SC, 224, 224), jnp.float32),
            pltpu.SemaphoreType.DMA((2,)),
            pltpu.SemaphoreType.DMA((2,)),
        ],
    )(sc_part, tc_full)


def kernel(x):
    tc_full = _tc_copy(x)
    sc_part = _sc_copy(x)
    return _merge(tc_full, sc_part)


# speedup vs baseline: 1.0018x; 1.0004x over previous
import functools

import jax
import jax.numpy as jnp
from jax import lax
from jax.experimental import pallas as pl
from jax.experimental.pallas import tpu as pltpu
from jax.experimental.pallas import tpu_sc as plsc

_B = 8
_C_OUT = 64
_STRIDE = 12

_G = 8
_NG_TC = 5
_C_TC = _G * _NG_TC
_C_SC = _C_OUT - _C_TC


_T = _B * _NG_TC
_RING = 8
_LOOK = 4


def _tc_kernel(x_ref, o_ref, buf, rsems, wsems):
    reads = [
        pltpu.make_async_copy(
            x_ref.at[t // _NG_TC, t % _NG_TC, :, 0],
            buf.at[t % _RING],
            rsems.at[t % _RING],
        )
        for t in range(_T)
    ]
    writes = [
        pltpu.make_async_copy(
            buf.at[t % _RING],
            o_ref.at[t // _NG_TC, pl.ds((t % _NG_TC) * _G, _G)],
            wsems.at[t % _RING],
        )
        for t in range(_T)
    ]
    for i in range(_T + _LOOK):
        if i < _T:
            if i >= _RING:
                writes[i - _RING].wait()
            reads[i].start()
        j = i - _LOOK
        if 0 <= j < _T:
            reads[j].wait()
            writes[j].start()
    for i in range(_T - _RING, _T):
        writes[i].wait()


def _tc_copy(x):
    xv = x.reshape(_B, _G, _G, _STRIDE, 224, 224)
    return pl.pallas_call(
        _tc_kernel,
        in_specs=[pl.BlockSpec(memory_space=pl.ANY)],
        out_specs=pl.BlockSpec(memory_space=pl.ANY),
        out_shape=jax.ShapeDtypeStruct((_B, _C_OUT, 224, 224), jnp.float32),
        scratch_shapes=[
            pltpu.VMEM((_RING, _G, 224, 224), jnp.float32),
            pltpu.SemaphoreType.DMA((_RING,)),
            pltpu.SemaphoreType.DMA((_RING,)),
        ],
    )(xv)


_NC = 2
_NS = 16
_NW = _NC * _NS
_N_SC = _B * _C_SC
_PER_W = _N_SC // _NW

_mesh = plsc.VectorSubcoreMesh(core_axis_name="c", subcore_axis_name="s")


@functools.partial(
    pl.kernel,
    out_type=jax.ShapeDtypeStruct((_B, _C_SC, 224, 224), jnp.float32),
    mesh=_mesh,
    scratch_types=[
        pltpu.VMEM((2, 224, 224), jnp.float32),
        pltpu.SemaphoreType.DMA((2,)),
        pltpu.SemaphoreType.DMA((2,)),
    ],
)
def _sc_copy(x_hbm, o_hbm, buf, isems, osems):
    wid = lax.axis_index("s") * _NC + lax.axis_index("c")

    def src(k):
        p = wid * _PER_W + k
        return x_hbm.at[p // _C_SC, (_C_TC + p % _C_SC) * _STRIDE]

    def dst(k):
        p = wid * _PER_W + k
        return o_hbm.at[p // _C_SC, p % _C_SC]

    cur_in = pltpu.async_copy(src(0), buf.at[0], isems.at[0])
    prev_out = None
    for k in range(_PER_W):
        s = k % 2
        cur_in.wait()
        if prev_out is not None:
            prev_out.wait()
        if k + 1 < _PER_W:
            cur_in = pltpu.async_copy(src(k + 1), buf.at[1 - s], isems.at[1 - s])
        prev_out = pltpu.async_copy(buf.at[s], dst(k), osems.at[s])
    prev_out.wait()


def _merge_kernel(sc_ref, tc_ref, o_ref, buf, rsems, wsems):
    reads = [
        pltpu.make_async_copy(sc_ref.at[b], buf.at[b % 2], rsems.at[b % 2])
        for b in range(_B)
    ]
    writes = [
        pltpu.make_async_copy(
            buf.at[b % 2], o_ref.at[b, pl.ds(_C_TC, _C_SC)], wsems.at[b % 2]
        )
        for b in range(_B)
    ]
    for i in range(_B + 1):
        if i < _B:
            if i >= 2:
                writes[i - 2].wait()
            reads[i].start()
        j = i - 1
        if 0 <= j < _B:
            reads[j].wait()
            writes[j].start()
    for i in range(_B - 2, _B):
        writes[i].wait()


def _merge(tc_full, sc_part):
    return pl.pallas_call(
        _merge_kernel,
        in_specs=[
            pl.BlockSpec(memory_space=pl.ANY),
            pl.BlockSpec(memory_space=pl.ANY),
        ],
        out_specs=pl.BlockSpec(memory_space=pl.ANY),
        out_shape=jax.ShapeDtypeStruct((_B, _C_OUT, 224, 224), jnp.float32),
        input_output_aliases={1: 0},
        scratch_shapes=[
            pltpu.VMEM((2, _C_SC, 224, 224), jnp.float32),
            pltpu.SemaphoreType.DMA((2,)),
            pltpu.SemaphoreType.DMA((2,)),
        ],
    )(sc_part, tc_full)


def kernel(x):
    sc_part = _sc_copy(x)
    tc_full = _tc_copy(x)
    return _merge(tc_full, sc_part)


# speedup vs baseline: 1.3294x; 1.3271x over previous
import jax
import jax.numpy as jnp
from jax.experimental import pallas as pl
from jax.experimental.pallas import tpu as pltpu

_R = 8
_T = 32
_L = 4


def _copy_kernel(x_ref, o_ref, buf, rsems, wsems):
    reads = [
        pltpu.make_async_copy(
            x_ref.at[i // 4, i % 4, :, 0], buf.at[i % _R], rsems.at[i % _R]
        )
        for i in range(_T)
    ]
    writes = [
        pltpu.make_async_copy(
            buf.at[i % _R], o_ref.at[i // 4, i % 4], wsems.at[i % _R]
        )
        for i in range(_T)
    ]
    for i in range(_T + _L):
        if i < _T:
            if i >= _R:
                writes[i - _R].wait()
            reads[i].start()
        j = i - _L
        if 0 <= j < _T:
            reads[j].wait()
            writes[j].start()
    for i in range(_T - _R, _T):
        writes[i].wait()


def kernel(x):
    xv = x.reshape(8, 4, 16, 12, 224, 224)
    out = pl.pallas_call(
        _copy_kernel,
        in_specs=[pl.BlockSpec(memory_space=pl.ANY)],
        out_specs=pl.BlockSpec(memory_space=pl.ANY),
        out_shape=jax.ShapeDtypeStruct((8, 4, 16, 224, 224), jnp.float32),
        scratch_shapes=[
            pltpu.VMEM((_R, 16, 224, 224), jnp.float32),
            pltpu.SemaphoreType.DMA((_R,)),
            pltpu.SemaphoreType.DMA((_R,)),
        ],
    )(xv)
    return out.reshape(8, 64, 224, 224)
